# baseline (speedup 1.0000x reference)
import jax
import jax.numpy as jnp
from jax.experimental import pallas as pl

N_A = 10000
N_D = 10000
D_IN = 256
HID = 64
HEADS = 8
HEAD_DIM = 64
CONV_OUT = HEADS * HEAD_DIM
OUT = 256
E = 160000


def _mm_kernel(x_ref, w_ref, o_ref):
    o_ref[...] = jnp.dot(x_ref[...], w_ref[...], preferred_element_type=jnp.float32)


def _mm(x, w):
    m, k = x.shape
    _, n = w.shape
    bm = 1000
    return pl.pallas_call(
        _mm_kernel,
        out_shape=jax.ShapeDtypeStruct((m, n), jnp.float32),
        grid=(m // bm,),
        in_specs=[
            pl.BlockSpec((bm, k), lambda i: (i, 0)),
            pl.BlockSpec((k, n), lambda i: (0, 0)),
        ],
        out_specs=pl.BlockSpec((bm, n), lambda i: (i, 0)),
    )(x, w)


def _att_aggregate(h_src, h_dst, att_s, att_d, src, dst, n_dst):
    a_s = (h_src * att_s[None, :, :]).sum(-1)
    a_d = (h_dst * att_d[None, :, :]).sum(-1)
    e = a_s[src] + a_d[dst]
    e = jax.nn.leaky_relu(e, 0.2)
    m = jax.ops.segment_max(e, dst, num_segments=n_dst)
    m = jnp.where(jnp.isfinite(m), m, 0.0)
    ex = jnp.exp(e - m[dst])
    s = jax.ops.segment_sum(ex, dst, num_segments=n_dst)
    alpha = ex / (s[dst] + 1e-16)
    out = jax.ops.segment_sum(h_src[src] * alpha[:, :, None], dst, num_segments=n_dst)
    return jax.nn.relu(out)


def kernel(x_author, x_document, edge_index_a2d, edge_index_d2d, W_a, b_a, W_d, b_d,
           P_a, pb_a, P_d, pb_d, att_s_a2d, att_d_a2d, att_s_d2d, att_d_d2d,
           Wk, bk, q, Wo, bo):
    x_a = _mm(x_author, W_a) + b_a
    x_d = _mm(x_document, W_d) + b_d
    h_a = (_mm(x_a, P_a) + pb_a).reshape(N_A, HEADS, HEAD_DIM)
    h_d = (_mm(x_d, P_d) + pb_d).reshape(N_D, HEADS, HEAD_DIM)
    out1 = _att_aggregate(h_a, h_d, att_s_a2d, att_d_a2d, edge_index_a2d[0], edge_index_a2d[1], N_D)
    out2 = _att_aggregate(h_d, h_d, att_s_d2d, att_d_d2d, edge_index_d2d[0], edge_index_d2d[1], N_D)
    z = jnp.stack([out1.reshape(N_D, CONV_OUT), out2.reshape(N_D, CONV_OUT)], axis=0)
    w = (jnp.tanh(z @ Wk + bk) @ q).mean(axis=1)
    beta = jax.nn.softmax(w)
    doc = (beta[:, None, None] * z).sum(0)
    return _mm(doc, Wo) + bo
